# baseline (speedup 1.0000x reference)
import functools

import jax
import jax.numpy as jnp
import numpy as np
from jax.experimental import pallas as pl
from jax.experimental.pallas import tpu as pltpu

N = 2048
K_NN = 16
EDGE_DIM = 128
NUM_LAYERS = 6
SIGMA = 1.0
N_HEADS = 8
FF_MULT = 4


def _layernorm(x, g, b):
    m = jnp.mean(x, axis=-1, keepdims=True)
    v = jnp.var(x, axis=-1, keepdims=True)
    return (x - m) / jnp.sqrt(v + 1e-5) * g + b


def _decoder_kernel(h_ref, ht_ref, wd_ref, bd_ref, out_ref):
    out_ref[...] = (h_ref[...] + ht_ref[...]) @ wd_ref[...] + bd_ref[0, 0]


def kernel(coords, Wq, Wk, Wv, Wo, ln1_g, ln1_b, W1, b1, W2, b2, ln2_g, ln2_b, Wd, bd):
    d = EDGE_DIM
    x2 = jnp.sum(coords * coords, axis=-1)
    d2 = x2[:, None] + x2[None, :] - 2.0 * (coords @ coords.T)
    dist = jnp.sqrt(jnp.maximum(d2, 1e-12))
    neg_kd, col_idx = jax.lax.top_k(-dist, K_NN)
    k_dist = -neg_kd
    sig = jnp.linspace(0.1 * SIGMA, 4.0 * SIGMA, d)
    vals = jnp.exp(-(k_dist[:, :, None] ** 2) / (2.0 * sig[None, None, :] ** 2))
    row = jnp.repeat(jnp.arange(N), K_NN)
    col = col_idx.reshape(-1)
    h = vals.reshape(N * K_NN, d)
    dh = d // N_HEADS
    for l in range(NUM_LAYERS):
        hn = _layernorm(h, ln1_g[l], ln1_b[l])
        q = (hn @ Wq[l]).reshape(N, K_NN, N_HEADS, dh)
        k_ = (hn @ Wk[l]).reshape(N, K_NN, N_HEADS, dh)
        v_ = (hn @ Wv[l]).reshape(N, K_NN, N_HEADS, dh)
        att = jnp.einsum('nqhd,nkhd->nhqk', q, k_) / np.sqrt(dh)
        att = jax.nn.softmax(att, axis=-1)
        o = jnp.einsum('nhqk,nkhd->nqhd', att, v_).reshape(N * K_NN, d)
        h = h + o @ Wo[l]
        hn = _layernorm(h, ln2_g[l], ln2_b[l])
        h = h + jax.nn.gelu(hn @ W1[l] + b1[l]) @ W2[l] + b2[l]
    keys = row * N + col
    tkeys = col * N + row
    order = jnp.argsort(keys)
    skeys = keys[order]
    pos = jnp.clip(jnp.searchsorted(skeys, tkeys), 0, keys.shape[0] - 1)
    found = (skeys[pos] == tkeys)[:, None]
    h_t = jnp.where(found, h[order][pos], 0.0)

    E = N * K_NN
    BLK = 4096
    logits = pl.pallas_call(
        _decoder_kernel,
        grid=(E // BLK,),
        in_specs=[
            pl.BlockSpec((BLK, d), lambda i: (i, 0)),
            pl.BlockSpec((BLK, d), lambda i: (i, 0)),
            pl.BlockSpec((d, 1), lambda i: (0, 0)),
            pl.BlockSpec((1, 1), lambda i: (0, 0), memory_space=pltpu.SMEM),
        ],
        out_specs=pl.BlockSpec((BLK, 1), lambda i: (i, 0)),
        out_shape=jax.ShapeDtypeStruct((E, 1), jnp.float32),
    )(h, h_t, Wd, bd.reshape(1, 1))
    return logits


# speedup vs baseline: 1.4496x; 1.4496x over previous
import functools

import jax
import jax.numpy as jnp
import numpy as np
from jax.experimental import pallas as pl
from jax.experimental.pallas import tpu as pltpu

N = 2048
K_NN = 16
EDGE_DIM = 128
NUM_LAYERS = 6
SIGMA = 1.0
N_HEADS = 8
FF_MULT = 4


def _layernorm(x, g, b):
    m = jnp.mean(x, axis=-1, keepdims=True)
    v = jnp.var(x, axis=-1, keepdims=True)
    return (x - m) / jnp.sqrt(v + 1e-5) * g + b


def _decoder_kernel(h_ref, ht_ref, wd_ref, bd_ref, out_ref):
    out_ref[...] = (h_ref[...] + ht_ref[...]) @ wd_ref[...] + bd_ref[0, 0]


def _topk_kernel(c_ref, ct_ref, kd_ref, ci_ref):
    cb = c_ref[...]
    ct = ct_ref[...]
    x2b = jnp.sum(cb * cb, axis=1, keepdims=True)
    x2a = jnp.sum(ct * ct, axis=0, keepdims=True)
    dot = jnp.dot(cb, ct, preferred_element_type=jnp.float32)
    d2 = (x2b + x2a) - 2.0 * dot
    dist = jnp.sqrt(jnp.maximum(d2, 1e-12))
    lane = jax.lax.broadcasted_iota(jnp.int32, dist.shape, 1)
    kd_cols = []
    ci_cols = []
    for _ in range(K_NN):
        m = jnp.min(dist, axis=1, keepdims=True)
        hit = dist == m
        idx = jnp.min(jnp.where(hit, lane, N), axis=1,
                      keepdims=True)
        kd_cols.append(m)
        ci_cols.append(idx)
        dist = jnp.where(lane == idx, 3.0e38, dist)
    kd_ref[...] = jnp.concatenate(kd_cols, axis=1)
    ci_ref[...] = jnp.concatenate(ci_cols, axis=1)


def _knn_topk(coords):
    BR = 256
    cpad = jnp.zeros((N, 8), jnp.float32).at[:, :3].set(coords)
    ct = cpad.T
    kd, ci = pl.pallas_call(
        _topk_kernel,
        grid=(N // BR,),
        in_specs=[
            pl.BlockSpec((BR, 8), lambda i: (i, 0)),
            pl.BlockSpec((8, N), lambda i: (0, 0)),
        ],
        out_specs=[
            pl.BlockSpec((BR, K_NN), lambda i: (i, 0)),
            pl.BlockSpec((BR, K_NN), lambda i: (i, 0)),
        ],
        out_shape=[
            jax.ShapeDtypeStruct((N, K_NN), jnp.float32),
            jax.ShapeDtypeStruct((N, K_NN), jnp.int32),
        ],
    )(cpad, ct)
    return kd, ci


def kernel(coords, Wq, Wk, Wv, Wo, ln1_g, ln1_b, W1, b1, W2, b2, ln2_g, ln2_b, Wd, bd):
    d = EDGE_DIM
    k_dist, col_idx = _knn_topk(coords)
    sig = jnp.linspace(0.1 * SIGMA, 4.0 * SIGMA, d)
    vals = jnp.exp(-(k_dist[:, :, None] ** 2) / (2.0 * sig[None, None, :] ** 2))
    row = jnp.repeat(jnp.arange(N), K_NN)
    col = col_idx.reshape(-1)
    h = vals.reshape(N * K_NN, d)
    dh = d // N_HEADS
    for l in range(NUM_LAYERS):
        hn = _layernorm(h, ln1_g[l], ln1_b[l])
        q = (hn @ Wq[l]).reshape(N, K_NN, N_HEADS, dh)
        k_ = (hn @ Wk[l]).reshape(N, K_NN, N_HEADS, dh)
        v_ = (hn @ Wv[l]).reshape(N, K_NN, N_HEADS, dh)
        att = jnp.einsum('nqhd,nkhd->nhqk', q, k_) / np.sqrt(dh)
        att = jax.nn.softmax(att, axis=-1)
        o = jnp.einsum('nhqk,nkhd->nqhd', att, v_).reshape(N * K_NN, d)
        h = h + o @ Wo[l]
        hn = _layernorm(h, ln2_g[l], ln2_b[l])
        h = h + jax.nn.gelu(hn @ W1[l] + b1[l]) @ W2[l] + b2[l]
    keys = row * N + col
    tkeys = col * N + row
    order = jnp.argsort(keys)
    skeys = keys[order]
    pos = jnp.clip(jnp.searchsorted(skeys, tkeys), 0, keys.shape[0] - 1)
    found = (skeys[pos] == tkeys)[:, None]
    h_t = jnp.where(found, h[order][pos], 0.0)

    E = N * K_NN
    BLK = 4096
    logits = pl.pallas_call(
        _decoder_kernel,
        grid=(E // BLK,),
        in_specs=[
            pl.BlockSpec((BLK, d), lambda i: (i, 0)),
            pl.BlockSpec((BLK, d), lambda i: (i, 0)),
            pl.BlockSpec((d, 1), lambda i: (0, 0)),
            pl.BlockSpec((1, 1), lambda i: (0, 0), memory_space=pltpu.SMEM),
        ],
        out_specs=pl.BlockSpec((BLK, 1), lambda i: (i, 0)),
        out_shape=jax.ShapeDtypeStruct((E, 1), jnp.float32),
    )(h, h_t, Wd, bd.reshape(1, 1))
    return logits


# speedup vs baseline: 1.7259x; 1.1906x over previous
import functools

import jax
import jax.numpy as jnp
import numpy as np
from jax import lax
from jax.experimental import pallas as pl
from jax.experimental.pallas import tpu as pltpu
from jax.experimental.pallas import tpu_sc as plsc

N = 2048
K_NN = 16
EDGE_DIM = 128
NUM_LAYERS = 6
SIGMA = 1.0
N_HEADS = 8
FF_MULT = 4


def _layernorm(x, g, b):
    m = jnp.mean(x, axis=-1, keepdims=True)
    v = jnp.var(x, axis=-1, keepdims=True)
    return (x - m) / jnp.sqrt(v + 1e-5) * g + b


def _decoder_kernel(h_ref, ht_ref, wd_ref, bd_ref, out_ref):
    out_ref[...] = (h_ref[...] + ht_ref[...]) @ wd_ref[...] + bd_ref[0, 0]


def _topk_kernel(c_ref, ct_ref, kd_ref, ci_ref):
    cb = c_ref[...]
    ct = ct_ref[...]
    x2b = jnp.sum(cb * cb, axis=1, keepdims=True)
    x2a = jnp.sum(ct * ct, axis=0, keepdims=True)
    dot = jnp.dot(cb, ct, preferred_element_type=jnp.float32)
    d2 = (x2b + x2a) - 2.0 * dot
    dist = jnp.sqrt(jnp.maximum(d2, 1e-12))
    lane = jax.lax.broadcasted_iota(jnp.int32, dist.shape, 1)
    kd_cols = []
    ci_cols = []
    for _ in range(K_NN):
        m = jnp.min(dist, axis=1, keepdims=True)
        hit = dist == m
        idx = jnp.min(jnp.where(hit, lane, N), axis=1,
                      keepdims=True)
        kd_cols.append(m)
        ci_cols.append(idx)
        dist = jnp.where(lane == idx, 3.0e38, dist)
    kd_ref[...] = jnp.concatenate(kd_cols, axis=1)
    ci_ref[...] = jnp.concatenate(ci_cols, axis=1)


_N_WORKERS = 32
_ROWS_PER_W = N // _N_WORKERS


def _sym_body(col_hbm, p_hbm, bdv_hbm, out_hbm, col_v, p_v, out_v, bdv_v):
    wid = lax.axis_index("s") * 2 + lax.axis_index("c")
    pltpu.sync_copy(col_hbm, col_v)
    pltpu.sync_copy(p_hbm, p_v)
    pltpu.sync_copy(bdv_hbm, bdv_v)
    base = wid * _ROWS_PER_W

    def row_body(r_local, carry):
        r = base + r_local
        c_vec = col_v[pl.ds(r * K_NN, K_NN)]
        p_vec = p_v[pl.ds(r * K_NN, K_NN)]
        acc = p_vec + bdv_v[...]
        rvec = jnp.full((K_NN,), 0, jnp.int32) + r
        for j in range(K_NN):
            idx = c_vec * K_NN + j
            cand_c = plsc.load_gather(col_v, [idx])
            cand_p = plsc.load_gather(p_v, [idx])
            acc = acc + jnp.where(cand_c == rvec, cand_p, 0.0)
        out_v[pl.ds(r_local * K_NN, K_NN)] = acc
        return carry

    lax.fori_loop(0, _ROWS_PER_W, row_body, 0)
    pltpu.sync_copy(out_v, out_hbm.at[pl.ds(base * K_NN, _ROWS_PER_W * K_NN)])


def _symmetrize(col_flat, p_flat, bd):
    E = N * K_NN
    bdv = jnp.broadcast_to(bd, (K_NN,)).astype(jnp.float32)
    mesh = plsc.VectorSubcoreMesh(core_axis_name="c", subcore_axis_name="s")
    f = functools.partial(
        pl.kernel,
        out_type=jax.ShapeDtypeStruct((E,), jnp.float32),
        mesh=mesh,
        scratch_types=[
            pltpu.VMEM((E,), jnp.int32),
            pltpu.VMEM((E,), jnp.float32),
            pltpu.VMEM((_ROWS_PER_W * K_NN,), jnp.float32),
            pltpu.VMEM((K_NN,), jnp.float32),
        ],
        compiler_params=pltpu.CompilerParams(needs_layout_passes=False),
    )(_sym_body)
    return f(col_flat, p_flat, bdv)


def _matvec_kernel(h_ref, wd_ref, out_ref):
    out_ref[...] = h_ref[...] @ wd_ref[...]


def _knn_topk(coords):
    BR = 256
    cpad = jnp.zeros((N, 8), jnp.float32).at[:, :3].set(coords)
    ct = cpad.T
    kd, ci = pl.pallas_call(
        _topk_kernel,
        grid=(N // BR,),
        in_specs=[
            pl.BlockSpec((BR, 8), lambda i: (i, 0)),
            pl.BlockSpec((8, N), lambda i: (0, 0)),
        ],
        out_specs=[
            pl.BlockSpec((BR, K_NN), lambda i: (i, 0)),
            pl.BlockSpec((BR, K_NN), lambda i: (i, 0)),
        ],
        out_shape=[
            jax.ShapeDtypeStruct((N, K_NN), jnp.float32),
            jax.ShapeDtypeStruct((N, K_NN), jnp.int32),
        ],
    )(cpad, ct)
    return kd, ci


def kernel(coords, Wq, Wk, Wv, Wo, ln1_g, ln1_b, W1, b1, W2, b2, ln2_g, ln2_b, Wd, bd):
    d = EDGE_DIM
    k_dist, col_idx = _knn_topk(coords)
    sig = jnp.linspace(0.1 * SIGMA, 4.0 * SIGMA, d)
    vals = jnp.exp(-(k_dist[:, :, None] ** 2) / (2.0 * sig[None, None, :] ** 2))
    row = jnp.repeat(jnp.arange(N), K_NN)
    col = col_idx.reshape(-1)
    h = vals.reshape(N * K_NN, d)
    dh = d // N_HEADS
    for l in range(NUM_LAYERS):
        hn = _layernorm(h, ln1_g[l], ln1_b[l])
        q = (hn @ Wq[l]).reshape(N, K_NN, N_HEADS, dh)
        k_ = (hn @ Wk[l]).reshape(N, K_NN, N_HEADS, dh)
        v_ = (hn @ Wv[l]).reshape(N, K_NN, N_HEADS, dh)
        att = jnp.einsum('nqhd,nkhd->nhqk', q, k_) / np.sqrt(dh)
        att = jax.nn.softmax(att, axis=-1)
        o = jnp.einsum('nhqk,nkhd->nqhd', att, v_).reshape(N * K_NN, d)
        h = h + o @ Wo[l]
        hn = _layernorm(h, ln2_g[l], ln2_b[l])
        h = h + jax.nn.gelu(hn @ W1[l] + b1[l]) @ W2[l] + b2[l]
    E = N * K_NN
    BLK = 4096
    p = pl.pallas_call(
        _matvec_kernel,
        grid=(E // BLK,),
        in_specs=[
            pl.BlockSpec((BLK, d), lambda i: (i, 0)),
            pl.BlockSpec((d, 1), lambda i: (0, 0)),
        ],
        out_specs=pl.BlockSpec((BLK, 1), lambda i: (i, 0)),
        out_shape=jax.ShapeDtypeStruct((E, 1), jnp.float32),
    )(h, Wd)
    logits = _symmetrize(col, p.reshape(E), bd)
    return logits.reshape(E, 1)
